# Initial kernel scaffold; baseline (speedup 1.0000x reference)
import jax
import jax.numpy as jnp
from jax.experimental import pallas as pl


def kernel(feats, coords, kernel, stride):
    raise NotImplementedError("write your pallas kernel here")


# speedup vs baseline: 1.3407x; 1.3407x over previous
import jax
import jax.numpy as jnp
from jax import lax
from jax.experimental import pallas as pl
from jax.experimental.pallas import tpu as pltpu

IN_C = 32
OUT_C = 32
NK = 32
GRID = 32
NUM_SEG = GRID * GRID * GRID
PAY = 48

BLK = 512
SEG_PER_TILE = NUM_SEG // 16


def _stage1_body(cext_ref, f_ref, kflat_ref, out_ref, key_ref):
    f = f_ref[...]
    c = cext_ref[...]
    kidx = c[:, 4:5]
    col = lax.broadcasted_iota(jnp.int32, (BLK, NK * IN_C), 1) // IN_C
    fe = jnp.concatenate([f] * NK, axis=1)
    expanded = jnp.where(col == kidx, fe, 0.0)
    res = jnp.dot(expanded, kflat_ref[...],
                  preferred_element_type=jnp.float32)
    lane = lax.broadcasted_iota(jnp.int32, (BLK, PAY), 1)
    padded = jnp.concatenate(
        [res, jnp.zeros((BLK, PAY - OUT_C), jnp.float32)], axis=1)
    out_ref[...] = jnp.where(lane == OUT_C, 1.0, padded)
    key_ref[...] = (c[:, 0:1] * GRID + c[:, 2:3]) * GRID + c[:, 3:4]


def _scatter_body(key_ref, pay_ref, acc_ref):
    @pl.when(pl.program_id(0) == 0)
    def _():
        acc_ref[...] = jnp.zeros((NUM_SEG, PAY), jnp.float32)

    def pt(n, carry):
        s = key_ref[n, 0]
        acc_ref[pl.ds(s, 1), :] += pay_ref[pl.ds(n, 1), :]
        return carry

    lax.fori_loop(0, BLK, pt, 0)


def _merge_body(a_ref, of_ref, oc_ref):
    s = a_ref[...]
    of_ref[...] = s[:, :OUT_C]
    occ = s[:, OUT_C:OUT_C + 1] > 0.5
    i = pl.program_id(0)
    seg = (lax.broadcasted_iota(jnp.int32, (SEG_PER_TILE, 1), 0)
           + i * SEG_PER_TILE)
    c0 = seg // (GRID * GRID)
    c2 = (seg // GRID) % GRID
    c3 = seg % GRID
    dec = jnp.concatenate(
        [c0, jnp.zeros((SEG_PER_TILE, 1), jnp.int32), c2, c3], axis=1)
    oc_ref[...] = jnp.where(occ, dec, 0)


def kernel(feats, coords, kernel, stride):
    n = feats.shape[0]
    npad = -(-n // BLK) * BLK
    nblk = npad // BLK
    pad = npad - n

    kidx = (coords[:, 1:2] // stride).astype(jnp.int32)
    cext = jnp.concatenate([coords, kidx], axis=1)
    cext = jnp.pad(cext, ((0, pad), (0, 0)))
    f_in = jnp.pad(feats, ((0, pad), (0, 0)))
    kflat = kernel.reshape(NK * IN_C, OUT_C)

    pay, key = pl.pallas_call(
        _stage1_body,
        grid=(nblk,),
        in_specs=[
            pl.BlockSpec((BLK, 5), lambda i: (i, 0)),
            pl.BlockSpec((BLK, IN_C), lambda i: (i, 0)),
            pl.BlockSpec((NK * IN_C, OUT_C), lambda i: (0, 0)),
        ],
        out_specs=[
            pl.BlockSpec((BLK, PAY), lambda i: (i, 0)),
            pl.BlockSpec((BLK, 1), lambda i: (i, 0)),
        ],
        out_shape=[
            jax.ShapeDtypeStruct((npad, PAY), jnp.float32),
            jax.ShapeDtypeStruct((npad, 1), jnp.int32),
        ],
    )(cext, f_in, kflat)

    acc = pl.pallas_call(
        _scatter_body,
        grid=(nblk,),
        in_specs=[
            pl.BlockSpec((BLK, 1), lambda i: (i, 0),
                         memory_space=pltpu.SMEM),
            pl.BlockSpec((BLK, PAY), lambda i: (i, 0)),
        ],
        out_specs=pl.BlockSpec((NUM_SEG, PAY), lambda i: (0, 0)),
        out_shape=jax.ShapeDtypeStruct((NUM_SEG, PAY), jnp.float32),
    )(key, pay)

    flat_feats, out_coords = pl.pallas_call(
        _merge_body,
        grid=(NUM_SEG // SEG_PER_TILE,),
        in_specs=[
            pl.BlockSpec((SEG_PER_TILE, PAY), lambda i: (i, 0)),
        ],
        out_specs=[
            pl.BlockSpec((SEG_PER_TILE, OUT_C), lambda i: (i, 0)),
            pl.BlockSpec((SEG_PER_TILE, 4), lambda i: (i, 0)),
        ],
        out_shape=[
            jax.ShapeDtypeStruct((NUM_SEG, OUT_C), jnp.float32),
            jax.ShapeDtypeStruct((NUM_SEG, 4), jnp.int32),
        ],
    )(acc)
    return flat_feats, out_coords
